# baseline (speedup 1.0000x reference)
import functools

import numpy as np

import jax
import jax.numpy as jnp
from jax import lax
from jax.experimental import pallas as pl
from jax.experimental.pallas import tpu as pltpu
from jax.experimental.pallas import tpu_sc as plsc

_B = 1024
_NBOX = 5
_PER = 25
_C = _NBOX * _PER
_G = 13
_HW = _G * _G
_NC, _NS, _L = 2, 16, 16
_NW = _NC * _NS
_EPW = _B // _NW
_LAMBDA_COORD = 5.0
_LAMBDA_NOOBJ = 0.5

_WOFF = [j * _L for j in range(10)] + [152]
_NWIN = len(_WOFF)


_mesh = plsc.VectorSubcoreMesh(
    core_axis_name="c", subcore_axis_name="s",
    num_cores=_NC, num_subcores=_NS,
)


@functools.partial(
    pl.kernel,
    out_type=jax.ShapeDtypeStruct((_NW, 16 * _L), jnp.float32),
    mesh=_mesh,
    compiler_params=pltpu.CompilerParams(use_tc_tiling_on_sc=False),
    scratch_types=[
        pltpu.VMEM((_C, _HW), jnp.float32),
        pltpu.VMEM((_C, _HW), jnp.float32),
        pltpu.VMEM((16 * _L,), jnp.float32),
        pltpu.VMEM((_L,), jnp.float32),
    ],
)
def _sc_loss(det_hbm, gt_hbm, tw_hbm, out_hbm, dbuf, gbuf, stage, cbuf):
    wid = lax.axis_index("s") * _NC + lax.axis_index("c")
    zero = jnp.zeros((_L,), jnp.float32)
    pltpu.sync_copy(tw_hbm, cbuf)

    def elem_body(e, acc):
        b = wid * _EPW + e
        pltpu.sync_copy(det_hbm.at[b], dbuf)
        pltpu.sync_copy(gt_hbm.at[b], gbuf)
        tw = cbuf[pl.ds(0, _L)]
        (coord, conf_obj, conf_all, cnt, cls, cntb) = acc
        z = tw * 0.0
        lcoord, lconf_obj, lconf_all, lcnt = z, z, z, z
        lcls = [z] * _NBOX
        lcntb = [z] * _NBOX
        for box in range(_NBOX):
            c4 = box * _PER + 4
            m = []
            for j in range(_NWIN):
                gv = gbuf[c4, pl.ds(_WOFF[j], _L)]
                dv = dbuf[c4, pl.ds(_WOFF[j], _L)]
                mh = 1.0 - jnp.abs(jnp.sign(gv - 1.0))
                if j == _NWIN - 1:
                    mh = mh * tw
                m.append(mh)
                df = dv - gv
                e2 = df * df
                lconf_obj = lconf_obj + e2 * mh
                lconf_all = lconf_all + (e2 * tw if j == _NWIN - 1 else e2)
                lcnt = lcnt + mh
                lcntb[box] = lcntb[box] + mh

            def chan_sum(k, s):
                for j in range(_NWIN):
                    dv = dbuf[k, pl.ds(_WOFF[j], _L)]
                    gv = gbuf[k, pl.ds(_WOFF[j], _L)]
                    df = dv - gv
                    s = s + df * df * m[j]
                return s

            for k in range(box * _PER, box * _PER + 4):
                lcoord = chan_sum(k, lcoord)
            for k in range(box * _PER + 5, (box + 1) * _PER):
                lcls[box] = chan_sum(k, lcls[box])
        coord = coord + lcoord
        conf_obj = conf_obj + lconf_obj
        conf_all = conf_all + lconf_all
        cnt = cnt + lcnt
        for box in range(_NBOX):
            cls[box] = cls[box] + lcls[box]
            cntb[box] = cntb[box] + lcntb[box]
        return (coord, conf_obj, conf_all, cnt, cls, cntb)

    acc0 = (zero, zero, zero, zero, [zero] * _NBOX, [zero] * _NBOX)
    coord, conf_obj, conf_all, cnt, cls, cntb = lax.fori_loop(
        0, _EPW, elem_body, acc0
    )

    rows = ([coord, conf_obj, conf_all, cnt] + list(cls) + list(cntb)
            + [zero, zero])
    for i, row in enumerate(rows):
        stage[pl.ds(i * _L, _L)] = row
    pltpu.sync_copy(stage, out_hbm.at[wid])


@jax.jit
def _detection_loss(det, gt):
    tw = np.array([0.0] * 8 + [1.0] * 8, np.float32)
    out = _sc_loss(det.reshape(_B, _C, _HW), gt.reshape(_B, _C, _HW), tw)
    acc = jnp.sum(out.reshape(_NW, 16, _L), axis=(0, 2))

    cnt = acc[3]
    total = float(_B * _NBOX * _HW)
    coord = jnp.where(cnt > 0, acc[0] / cnt, 0.0)
    conf_obj = jnp.where(cnt > 0, acc[1] / cnt, 0.0)
    obj_loss = _LAMBDA_COORD * coord + conf_obj
    noobj_cnt = total - cnt
    no_obj_loss = _LAMBDA_NOOBJ * jnp.where(
        noobj_cnt > 0, (acc[2] - acc[1]) / noobj_cnt, 0.0
    )
    confidence = 0.0
    for b in range(_NBOX):
        cnt_b = acc[9 + b] * 20.0
        confidence = confidence + jnp.where(cnt_b > 0, acc[4 + b] / cnt_b, 0.0)
    loss = obj_loss + no_obj_loss + confidence
    return (loss, obj_loss, no_obj_loss, confidence)


def kernel(detection_result, gt_grid):
    return _detection_loss(detection_result, gt_grid)


# speedup vs baseline: 1.0029x; 1.0029x over previous
import functools

import numpy as np

import jax
import jax.numpy as jnp
from jax import lax
from jax.experimental import pallas as pl
from jax.experimental.pallas import tpu as pltpu
from jax.experimental.pallas import tpu_sc as plsc

_B = 1024
_NBOX = 5
_PER = 25
_C = _NBOX * _PER
_G = 13
_HW = _G * _G
_NC, _NS, _L = 2, 16, 16
_NW = _NC * _NS
_EPW = _B // _NW
_LAMBDA_COORD = 5.0
_LAMBDA_NOOBJ = 0.5

_WOFF = [j * _L for j in range(10)] + [_HW - _L]
_NWIN = len(_WOFF)


_mesh = plsc.VectorSubcoreMesh(
    core_axis_name="c", subcore_axis_name="s",
    num_cores=_NC, num_subcores=_NS,
)


@functools.partial(
    pl.kernel,
    out_type=jax.ShapeDtypeStruct((_NW, 16 * _L), jnp.float32),
    mesh=_mesh,
    compiler_params=pltpu.CompilerParams(use_tc_tiling_on_sc=False),
    scratch_types=[
        pltpu.VMEM((_C, _HW), jnp.float32),
        pltpu.VMEM((_C, _HW), jnp.float32),
        pltpu.VMEM((16 * _L,), jnp.float32),
        pltpu.VMEM((_L,), jnp.float32),
    ],
)
def _sc_loss(det_hbm, gt_hbm, tw_hbm, out_hbm, dbuf, gbuf, stage, cbuf):
    wid = lax.axis_index("s") * _NC + lax.axis_index("c")
    zero = jnp.zeros((_L,), jnp.float32)
    pltpu.sync_copy(tw_hbm, cbuf)

    def elem_body(e, acc):
        b = wid * _EPW + e
        pltpu.sync_copy(det_hbm.at[b], dbuf)
        pltpu.sync_copy(gt_hbm.at[b], gbuf)
        tw = cbuf[pl.ds(0, _L)]
        (coord, conf_obj, conf_all, cnt, cls, cntb) = acc
        z = tw * 0.0
        lcoord, lconf_obj, lconf_all, lcnt = z, z, z, z
        lcls = [z] * _NBOX
        lcntb = [z] * _NBOX
        for box in range(_NBOX):
            c4 = box * _PER + 4
            m = []
            for j in range(_NWIN):
                gv = gbuf[c4, pl.ds(_WOFF[j], _L)]
                dv = dbuf[c4, pl.ds(_WOFF[j], _L)]
                mh = 1.0 - jnp.abs(jnp.sign(gv - 1.0))
                if j == _NWIN - 1:
                    mh = mh * tw
                m.append(mh)
                df = dv - gv
                e2 = df * df
                lconf_obj = lconf_obj + e2 * mh
                lconf_all = lconf_all + (e2 * tw if j == _NWIN - 1 else e2)
                lcnt = lcnt + mh
                lcntb[box] = lcntb[box] + mh

            def chan_sum(k, s):
                for j in range(_NWIN):
                    dv = dbuf[k, pl.ds(_WOFF[j], _L)]
                    gv = gbuf[k, pl.ds(_WOFF[j], _L)]
                    df = dv - gv
                    s = s + df * df * m[j]
                return s

            for k in range(box * _PER, box * _PER + 4):
                lcoord = chan_sum(k, lcoord)
            for k in range(box * _PER + 5, (box + 1) * _PER):
                lcls[box] = chan_sum(k, lcls[box])
        coord = coord + lcoord
        conf_obj = conf_obj + lconf_obj
        conf_all = conf_all + lconf_all
        cnt = cnt + lcnt
        for box in range(_NBOX):
            cls[box] = cls[box] + lcls[box]
            cntb[box] = cntb[box] + lcntb[box]
        return (coord, conf_obj, conf_all, cnt, cls, cntb)

    acc0 = (zero, zero, zero, zero, [zero] * _NBOX, [zero] * _NBOX)
    coord, conf_obj, conf_all, cnt, cls, cntb = lax.fori_loop(
        0, _EPW, elem_body, acc0
    )

    rows = ([coord, conf_obj, conf_all, cnt] + list(cls) + list(cntb)
            + [zero, zero])
    for i, row in enumerate(rows):
        stage[pl.ds(i * _L, _L)] = row
    pltpu.sync_copy(stage, out_hbm.at[wid])


@jax.jit
def _detection_loss(det, gt):
    tw = np.array([0.0] * 7 + [1.0] * 9, np.float32)
    out = _sc_loss(det.reshape(_B, _C, _HW), gt.reshape(_B, _C, _HW), tw)
    acc = jnp.sum(out.reshape(_NW, 16, _L), axis=(0, 2))

    cnt = acc[3]
    total = float(_B * _NBOX * _HW)
    coord = jnp.where(cnt > 0, acc[0] / cnt, 0.0)
    conf_obj = jnp.where(cnt > 0, acc[1] / cnt, 0.0)
    obj_loss = _LAMBDA_COORD * coord + conf_obj
    noobj_cnt = total - cnt
    no_obj_loss = _LAMBDA_NOOBJ * jnp.where(
        noobj_cnt > 0, (acc[2] - acc[1]) / noobj_cnt, 0.0
    )
    confidence = 0.0
    for b in range(_NBOX):
        cnt_b = acc[9 + b] * 20.0
        confidence = confidence + jnp.where(cnt_b > 0, acc[4 + b] / cnt_b, 0.0)
    loss = obj_loss + no_obj_loss + confidence
    return (loss, obj_loss, no_obj_loss, confidence)


def kernel(detection_result, gt_grid):
    return _detection_loss(detection_result, gt_grid)


# speedup vs baseline: 3.6796x; 3.6689x over previous
import numpy as np

import jax
import jax.numpy as jnp
from jax.experimental import pallas as pl
from jax.experimental.pallas import tpu as pltpu

_B = 1024
_NBOX = 5
_PER = 25
_C = _NBOX * _PER
_HW = 169
_R = _B * _HW
_STEPS = 16
_BLK_R = _R // _STEPS
_LAMBDA_COORD = 5.0
_LAMBDA_NOOBJ = 0.5


def _patterns():
    c = np.arange(_C)
    box, k = c // _PER, c % _PER
    spread = np.zeros((_C, _C), np.float32)
    spread[box * _PER + 4, c] = 1.0
    p_me = np.zeros((_C, 128), np.float32)
    p_me[:, 0] = (k < 4)
    p_me[:, 1] = (k == 4)
    for b in range(_NBOX):
        p_me[:, 4 + b] = (box == b) & (k >= 5)
    p_err = np.zeros((_C, 128), np.float32)
    p_err[:, 2] = (k == 4)
    p_m = np.zeros((_C, 128), np.float32)
    p_m[:, 3] = (k == 4)
    for b in range(_NBOX):
        p_m[:, 9 + b] = (c == b * _PER + 4)
    return spread, p_me, p_err, p_m


_SPREAD, _P_ME, _P_ERR, _P_M = _patterns()


def _loss_body(det_ref, gt_ref, spread_ref, pme_ref, perr_ref, pm_ref, acc_ref):
    step = pl.program_id(0)

    @pl.when(step == 0)
    def _init():
        acc_ref[...] = jnp.zeros_like(acc_ref)

    d = det_ref[...].astype(jnp.float32)
    g = gt_ref[...].astype(jnp.float32)
    err = (d - g) ** 2
    ones = (g == 1.0).astype(jnp.float32)
    mask = jnp.dot(ones, spread_ref[...])
    me = err * mask
    part = (
        jnp.dot(me, pme_ref[...])
        + jnp.dot(err, perr_ref[...])
        + jnp.dot(mask, pm_ref[...])
    )
    acc_ref[...] += jnp.sum(part, axis=0, keepdims=True)


@jax.jit
def _detection_loss(det, gt):
    det2 = (
        jnp.transpose(det.reshape(_B, _C, _HW), (0, 2, 1))
        .reshape(_R, _C)
        .astype(jnp.bfloat16)
    )
    gt2 = (
        jnp.transpose(gt.reshape(_B, _C, _HW), (0, 2, 1))
        .reshape(_R, _C)
        .astype(jnp.bfloat16)
    )
    data_spec = pl.BlockSpec((_BLK_R, _C), lambda i: (i, 0))
    const_spec = pl.BlockSpec((_C, _C), lambda i: (0, 0))
    pat_spec = pl.BlockSpec((_C, 128), lambda i: (0, 0))
    acc = pl.pallas_call(
        _loss_body,
        grid=(_STEPS,),
        in_specs=[data_spec, data_spec, const_spec, pat_spec, pat_spec, pat_spec],
        out_specs=pl.BlockSpec((1, 128), lambda i: (0, 0)),
        out_shape=jax.ShapeDtypeStruct((1, 128), jnp.float32),
    )(det2, gt2, _SPREAD, _P_ME, _P_ERR, _P_M)[0]

    cnt = acc[3]
    total = float(_R * _NBOX)
    coord = jnp.where(cnt > 0, acc[0] / cnt, 0.0)
    conf_obj = jnp.where(cnt > 0, acc[1] / cnt, 0.0)
    obj_loss = _LAMBDA_COORD * coord + conf_obj
    noobj_cnt = total - cnt
    no_obj_loss = _LAMBDA_NOOBJ * jnp.where(
        noobj_cnt > 0, (acc[2] - acc[1]) / noobj_cnt, 0.0
    )
    confidence = 0.0
    for b in range(_NBOX):
        cnt_b = acc[9 + b] * 20.0
        confidence = confidence + jnp.where(cnt_b > 0, acc[4 + b] / cnt_b, 0.0)
    loss = obj_loss + no_obj_loss + confidence
    return (loss, obj_loss, no_obj_loss, confidence)


def kernel(detection_result, gt_grid):
    return _detection_loss(detection_result, gt_grid)
